# baseline (speedup 1.0000x reference)
import functools

import jax
import jax.numpy as jnp
from jax import lax
from jax.experimental import pallas as pl
from jax.experimental.pallas import tpu as pltpu

_NE = 1024
_D = 64
_HW = 576
_B = 64
_N = _B * _HW


def _vq_body(z_ref, emb_ref, embm2_ref, idx_ref, qt_ref, loss_ref):
    x = z_ref[0]
    emb = emb_ref[...]
    a2 = jnp.sum(x * x, axis=1, keepdims=True)
    b2 = jnp.sum(emb * emb, axis=1)[None, :]
    ab2 = lax.dot_general(x, embm2_ref[...], (((1,), (1,)), ((), ())),
                          preferred_element_type=jnp.float32)
    sq = (a2 + b2) + ab2
    d = jnp.sqrt(jnp.maximum(sq, 0.0))
    dmin = jnp.min(d, axis=1, keepdims=True)
    idx = jnp.argmin(d, axis=1).astype(jnp.int32)
    idx_ref[0, 0, :] = idx
    ji = lax.broadcasted_iota(jnp.int32, (_HW, _NE), 1)
    onehot = jnp.where(ji == idx[:, None], 1.0, 0.0)
    qt = lax.dot_general(emb, onehot, (((0,), (1,)), ((), ())),
                         preferred_element_type=jnp.float32)
    qt_ref[0] = qt
    part = jnp.sum(dmin * dmin)
    @pl.when(pl.program_id(0) == 0)
    def _():
        loss_ref[0, 0] = 0.0
    loss_ref[0, 0] += part


@jax.jit
def kernel(z, embeddings):
    zf = z.reshape(_B, _HW, _D)
    idx3, qt, loss_acc = pl.pallas_call(
        _vq_body,
        grid=(_B,),
        in_specs=[
            pl.BlockSpec((1, _HW, _D), lambda i: (i, 0, 0)),
            pl.BlockSpec((_NE, _D), lambda i: (0, 0)),
            pl.BlockSpec((_NE, _D), lambda i: (0, 0)),
        ],
        out_specs=[
            pl.BlockSpec((1, 1, _HW), lambda i: (i, 0, 0)),
            pl.BlockSpec((1, _D, _HW), lambda i: (i, 0, 0)),
            pl.BlockSpec(memory_space=pltpu.SMEM, block_shape=(1, 1),
                         index_map=lambda i: (0, 0)),
        ],
        out_shape=[
            jax.ShapeDtypeStruct((_B, 1, _HW), jnp.int32),
            jax.ShapeDtypeStruct((_B, _D, _HW), jnp.float32),
            jax.ShapeDtypeStruct((1, 1), jnp.float32),
        ],
        compiler_params=pltpu.CompilerParams(
            dimension_semantics=("arbitrary",)),
    )(zf, embeddings, embeddings * jnp.float32(-2.0))
    quantized = qt.reshape(_B, _D, 24, 24)
    indices = idx3.reshape(_B, 1, 24, 24)
    loss = (loss_acc[0, 0] / jnp.float32(_N * _D)) * jnp.float32(1.25)
    return quantized, indices, loss


# speedup vs baseline: 1.1298x; 1.1298x over previous
import functools

import jax
import jax.numpy as jnp
from jax import lax
from jax.experimental import pallas as pl
from jax.experimental.pallas import tpu as pltpu

_NE = 1024
_D = 64
_HW = 576
_B = 64
_N = _B * _HW


def _vq_body(z_ref, emb_ref, embm2_ref, idx_ref, qt_ref, loss_ref):
    x = z_ref[0]
    emb = emb_ref[...]
    a2 = jnp.sum(x * x, axis=1, keepdims=True)
    b2 = jnp.sum(emb * emb, axis=1)[None, :]
    ab2 = lax.dot_general(x, embm2_ref[...], (((1,), (1,)), ((), ())),
                          preferred_element_type=jnp.float32)
    sq = (a2 + b2) + ab2
    d = jnp.sqrt(jnp.maximum(sq, 0.0))
    dmin = jnp.min(d, axis=1, keepdims=True)
    ji = lax.broadcasted_iota(jnp.int32, (_HW, _NE), 1)
    idx = jnp.min(jnp.where(d == dmin, ji, jnp.int32(2**30)), axis=1)
    idx_ref[0, 0, :] = idx
    onehot = jnp.where(ji == idx[:, None], 1.0, 0.0)
    qt = lax.dot_general(emb, onehot, (((0,), (1,)), ((), ())),
                         preferred_element_type=jnp.float32)
    qt_ref[0] = qt
    part = jnp.sum(dmin * dmin)
    @pl.when(pl.program_id(0) == 0)
    def _():
        loss_ref[0, 0] = 0.0
    loss_ref[0, 0] += part


@jax.jit
def kernel(z, embeddings):
    zf = z.reshape(_B, _HW, _D)
    idx3, qt, loss_acc = pl.pallas_call(
        _vq_body,
        grid=(_B,),
        in_specs=[
            pl.BlockSpec((1, _HW, _D), lambda i: (i, 0, 0)),
            pl.BlockSpec((_NE, _D), lambda i: (0, 0)),
            pl.BlockSpec((_NE, _D), lambda i: (0, 0)),
        ],
        out_specs=[
            pl.BlockSpec((1, 1, _HW), lambda i: (i, 0, 0)),
            pl.BlockSpec((1, _D, _HW), lambda i: (i, 0, 0)),
            pl.BlockSpec(memory_space=pltpu.SMEM, block_shape=(1, 1),
                         index_map=lambda i: (0, 0)),
        ],
        out_shape=[
            jax.ShapeDtypeStruct((_B, 1, _HW), jnp.int32),
            jax.ShapeDtypeStruct((_B, _D, _HW), jnp.float32),
            jax.ShapeDtypeStruct((1, 1), jnp.float32),
        ],
        compiler_params=pltpu.CompilerParams(
            dimension_semantics=("arbitrary",)),
    )(zf, embeddings, embeddings * jnp.float32(-2.0))
    quantized = qt.reshape(_B, _D, 24, 24)
    indices = idx3.reshape(_B, 1, 24, 24)
    loss = (loss_acc[0, 0] / jnp.float32(_N * _D)) * jnp.float32(1.25)
    return quantized, indices, loss
